# baseline (speedup 1.0000x reference)
import jax
import jax.numpy as jnp
from jax.experimental import pallas as pl
from jax.experimental.pallas import tpu as pltpu

B, D, H, W = 4, 192, 224, 224
NSTREAM = 2
DC = 8
RPS = NSTREAM * DC
ND = D // RPS
EPS = 1e-8


def _loss_kernel(*refs):
    s_refs = refs[0:NSTREAM]
    t_refs = refs[NSTREAM:2 * NSTREAM]
    ox_ref, m_ref, c_ref, out_ref, st_ref, ss_ref, tt_ref, ls_ref, cn_ref = \
        refs[2 * NSTREAM:]
    b = pl.program_id(0)
    k = pl.program_id(1)

    @pl.when(k == 0)
    def _init():
        st_ref[...] = jnp.zeros_like(st_ref)
        ss_ref[...] = jnp.zeros_like(ss_ref)
        tt_ref[...] = jnp.zeros_like(tt_ref)

    st_acc = ss_acc = tt_acc = None
    for si in range(NSTREAM):
        s = s_refs[si][0]
        t = t_refs[si][0] - c_ref[0, si * DC:(si + 1) * DC]
        for g in range(DC // 8):
            sl = slice(8 * g, 8 * (g + 1))
            sg, tg = s[sl], t[sl]
            if st_acc is None:
                st_acc, ss_acc, tt_acc = sg * tg, sg * sg, tg * tg
            else:
                st_acc += sg * tg
                ss_acc += sg * sg
                tt_acc += tg * tg
    st_ref[...] += st_acc
    ss_ref[...] += ss_acc
    tt_ref[...] += tt_acc

    @pl.when(k == ND - 1)
    def _per_image():
        st = jnp.sum(st_ref[...], axis=0)
        ss = jnp.sum(ss_ref[...], axis=0)
        tt = jnp.sum(tt_ref[...], axis=0)
        s_n = jnp.maximum(jnp.sqrt(ss), EPS)
        t_n = jnp.maximum(jnp.sqrt(tt), EPS)
        loss = 1.0 - st / (s_n * t_n)
        valid = (ox_ref[0, 0] != 0.0) & jnp.logical_not(m_ref[0])
        vf = valid.astype(jnp.float32)
        ls_ref[b] = jnp.sum(loss * vf)
        cn_ref[b] = jnp.sum(vf)

    @pl.when((k == ND - 1) & (b == B - 1))
    def _final():
        num = 0.0
        den = 0.0
        total = 0.0
        for i in range(B):
            cn = cn_ref[i]
            hv = jnp.where(cn > 0.0, 1.0, 0.0)
            num += hv * ls_ref[i] / jnp.maximum(cn, 1.0)
            den += hv
            total += cn
        mean = num / jnp.maximum(den, 1.0)
        out_ref[0] = jnp.where(total == 0.0, 0.0, mean)


def _feat_spec(si):
    return pl.BlockSpec((1, DC, H, W),
                        lambda b, k, si=si: (b, NSTREAM * k + si, 0, 0))


def kernel(student_feats, teacher_feats, mask, original_x, center):
    c = center.reshape(ND, RPS, 1, 1)

    out = pl.pallas_call(
        _loss_kernel,
        grid=(B, ND),
        in_specs=(
            [_feat_spec(si) for si in range(NSTREAM)]
            + [_feat_spec(si) for si in range(NSTREAM)]
            + [
                pl.BlockSpec((1, 1, H, W), lambda b, k: (b, 0, 0, 0)),
                pl.BlockSpec((1, H, W), lambda b, k: (b, 0, 0)),
                pl.BlockSpec((1, RPS, 1, 1), lambda b, k: (k, 0, 0, 0)),
            ]
        ),
        out_specs=pl.BlockSpec(memory_space=pltpu.SMEM),
        out_shape=jax.ShapeDtypeStruct((1,), jnp.float32),
        scratch_shapes=[
            pltpu.VMEM((8, H, W), jnp.float32),
            pltpu.VMEM((8, H, W), jnp.float32),
            pltpu.VMEM((8, H, W), jnp.float32),
            pltpu.SMEM((B,), jnp.float32),
            pltpu.SMEM((B,), jnp.float32),
        ],
    )(*([student_feats] * NSTREAM + [teacher_feats] * NSTREAM
        + [original_x, mask, c]))
    return out[0]


# speedup vs baseline: 1.1252x; 1.1252x over previous
import jax
import jax.numpy as jnp
from jax.experimental import pallas as pl
from jax.experimental.pallas import tpu as pltpu

B, D, H, W = 4, 192, 224, 224
NSTREAM = 2
DC = 12
RPS = NSTREAM * DC
ND = D // RPS
EPS = 1e-8


def _loss_kernel(*refs):
    s_refs = refs[0:NSTREAM]
    t_refs = refs[NSTREAM:2 * NSTREAM]
    ox_ref, m_ref, c_ref, out_ref, st_ref, ss_ref, tt_ref, ls_ref, cn_ref = \
        refs[2 * NSTREAM:]
    b = pl.program_id(0)
    k = pl.program_id(1)

    @pl.when(k == 0)
    def _init():
        st_ref[...] = jnp.zeros_like(st_ref)
        ss_ref[...] = jnp.zeros_like(ss_ref)
        tt_ref[...] = jnp.zeros_like(tt_ref)

    st_acc = ss_acc = tt_acc = None
    for si in range(NSTREAM):
        s = s_refs[si][0]
        t = t_refs[si][0] - c_ref[0, si * DC:(si + 1) * DC]
        for g in range(DC // 4):
            sl = slice(4 * g, 4 * (g + 1))
            sg, tg = s[sl], t[sl]
            if st_acc is None:
                st_acc, ss_acc, tt_acc = sg * tg, sg * sg, tg * tg
            else:
                st_acc += sg * tg
                ss_acc += sg * sg
                tt_acc += tg * tg
    st_ref[...] += st_acc
    ss_ref[...] += ss_acc
    tt_ref[...] += tt_acc

    @pl.when(k == ND - 1)
    def _per_image():
        st = jnp.sum(st_ref[...], axis=0)
        ss = jnp.sum(ss_ref[...], axis=0)
        tt = jnp.sum(tt_ref[...], axis=0)
        s_n = jnp.maximum(jnp.sqrt(ss), EPS)
        t_n = jnp.maximum(jnp.sqrt(tt), EPS)
        loss = 1.0 - st / (s_n * t_n)
        valid = (ox_ref[0, 0] != 0.0) & jnp.logical_not(m_ref[0])
        vf = valid.astype(jnp.float32)
        ls_ref[b] = jnp.sum(loss * vf)
        cn_ref[b] = jnp.sum(vf)

    @pl.when((k == ND - 1) & (b == B - 1))
    def _final():
        num = 0.0
        den = 0.0
        total = 0.0
        for i in range(B):
            cn = cn_ref[i]
            hv = jnp.where(cn > 0.0, 1.0, 0.0)
            num += hv * ls_ref[i] / jnp.maximum(cn, 1.0)
            den += hv
            total += cn
        mean = num / jnp.maximum(den, 1.0)
        out_ref[0] = jnp.where(total == 0.0, 0.0, mean)


def _feat_spec(si):
    return pl.BlockSpec((1, DC, H, W),
                        lambda b, k, si=si: (b, NSTREAM * k + si, 0, 0))


def kernel(student_feats, teacher_feats, mask, original_x, center):
    c = center.reshape(ND, RPS, 1, 1)

    out = pl.pallas_call(
        _loss_kernel,
        grid=(B, ND),
        in_specs=(
            [_feat_spec(si) for si in range(NSTREAM)]
            + [_feat_spec(si) for si in range(NSTREAM)]
            + [
                pl.BlockSpec((1, 1, H, W), lambda b, k: (b, 0, 0, 0)),
                pl.BlockSpec((1, H, W), lambda b, k: (b, 0, 0)),
                pl.BlockSpec((1, RPS, 1, 1), lambda b, k: (k, 0, 0, 0)),
            ]
        ),
        out_specs=pl.BlockSpec(memory_space=pltpu.SMEM),
        out_shape=jax.ShapeDtypeStruct((1,), jnp.float32),
        scratch_shapes=[
            pltpu.VMEM((4, H, W), jnp.float32),
            pltpu.VMEM((4, H, W), jnp.float32),
            pltpu.VMEM((4, H, W), jnp.float32),
            pltpu.SMEM((B,), jnp.float32),
            pltpu.SMEM((B,), jnp.float32),
        ],
    )(*([student_feats] * NSTREAM + [teacher_feats] * NSTREAM
        + [original_x, mask, c]))
    return out[0]
